# baseline (speedup 1.0000x reference)
import jax
import jax.numpy as jnp
from jax.experimental import pallas as pl
from jax.experimental.pallas import tpu as pltpu

N = 10000
F = 128
NS = 5
RB = 40
NSTEP = N // (NS * RB)


def _body(*refs):
    adj_refs = refs[:NS]
    (x_ref, W1_ref, b1_ref, W2_ref, b2_ref, Wc_ref, bc_ref,
     out_ref, u_ref, s1_ref, v_ref) = refs[NS:]
    i = pl.program_id(0)

    @pl.when(i == 0)
    def _init():
        s1_ref[...] = jnp.dot(x_ref[...], W1_ref[...],
                              preferred_element_type=jnp.float32)
        v_ref[...] = jnp.dot(W2_ref[...], Wc_ref[...],
                             preferred_element_type=jnp.float32)

    @pl.when(i < NSTEP)
    def _phase1():
        base = i * NS * RB
        for s in range(NS):
            z = jnp.dot(adj_refs[s][...], s1_ref[...],
                        preferred_element_type=jnp.float32)
            z = jnp.maximum(z + b1_ref[...], 0.0)
            u_ref[pl.ds(base + s * RB, RB), :] = jnp.dot(
                z, v_ref[...], preferred_element_type=jnp.float32)

    @pl.when(i >= NSTEP)
    def _phase2():
        c = jnp.dot(b2_ref[...], Wc_ref[...],
                    preferred_element_type=jnp.float32) + bc_ref[...]
        for s in range(NS):
            out_ref[s * RB:(s + 1) * RB, :] = jnp.dot(
                adj_refs[s][...], u_ref[...],
                preferred_element_type=jnp.float32) + c


def kernel(adj, x, W1, b1, W2, b2, Wc, bc, We1, be1, We2, be2, Wfc, bfc):
    del We1, be1, We2, be2, Wfc, bfc
    b1_2d = b1.reshape(1, F)
    b2_2d = b2.reshape(1, F)
    bc_2d = bc.reshape(1, 1)

    adj_specs = [
        pl.BlockSpec((RB, N), lambda i, s=s: (NS * (i % NSTEP) + s, 0))
        for s in range(NS)
    ]
    label = pl.pallas_call(
        _body,
        grid=(2 * NSTEP,),
        in_specs=adj_specs + [
            pl.BlockSpec((N, F), lambda i: (0, 0)),
            pl.BlockSpec((F, F), lambda i: (0, 0)),
            pl.BlockSpec((1, F), lambda i: (0, 0)),
            pl.BlockSpec((F, F), lambda i: (0, 0)),
            pl.BlockSpec((1, F), lambda i: (0, 0)),
            pl.BlockSpec((F, 1), lambda i: (0, 0)),
            pl.BlockSpec((1, 1), lambda i: (0, 0)),
        ],
        out_specs=pl.BlockSpec(
            (NS * RB, 1), lambda i: (jnp.where(i < NSTEP, 0, i - NSTEP), 0)),
        out_shape=jax.ShapeDtypeStruct((N, 1), jnp.float32),
        scratch_shapes=[
            pltpu.VMEM((N, 1), jnp.float32),
            pltpu.VMEM((N, F), jnp.float32),
            pltpu.VMEM((F, 1), jnp.float32),
        ],
    )(*([adj] * NS), x, W1, b1_2d, W2, b2_2d, Wc, bc_2d)
    return label


# speedup vs baseline: 1.0784x; 1.0784x over previous
import jax
import jax.numpy as jnp
from jax.experimental import pallas as pl
from jax.experimental.pallas import tpu as pltpu

N = 10000
F = 128
NS = 2
RB = 200
NSTEP = N // (NS * RB)


def _p1_body(adjA_ref, adjB_ref, x_ref, W1_ref, b1_ref, W2_ref, Wc_ref,
             u_ref, s1_ref, v_ref):
    i = pl.program_id(0)

    @pl.when(i == 0)
    def _init():
        s1_ref[...] = jnp.dot(x_ref[...], W1_ref[...],
                              preferred_element_type=jnp.float32)
        v_ref[...] = jnp.dot(W2_ref[...], Wc_ref[...],
                             preferred_element_type=jnp.float32)

    for s, a_ref in enumerate((adjA_ref, adjB_ref)):
        z = jnp.dot(a_ref[...], s1_ref[...],
                    preferred_element_type=jnp.float32)
        z = jnp.maximum(z + b1_ref[...], 0.0)
        u_ref[s * RB:(s + 1) * RB, :] = jnp.dot(
            z, v_ref[...], preferred_element_type=jnp.float32)


def _p2_body(adjA_ref, adjB_ref, u_ref, c_ref, out_ref):
    for s, a_ref in enumerate((adjA_ref, adjB_ref)):
        out_ref[s * RB:(s + 1) * RB, :] = jnp.dot(
            a_ref[...], u_ref[...],
            preferred_element_type=jnp.float32) + c_ref[0, 0]


def kernel(adj, x, W1, b1, W2, b2, Wc, bc, We1, be1, We2, be2, Wfc, bfc):
    del We1, be1, We2, be2, Wfc, bfc
    b1_2d = b1.reshape(1, F)

    adj_specs = [
        pl.BlockSpec((RB, N), lambda i, s=s: (NS * i + s, 0))
        for s in range(NS)
    ]
    u = pl.pallas_call(
        _p1_body,
        grid=(NSTEP,),
        in_specs=adj_specs + [
            pl.BlockSpec((N, F), lambda i: (0, 0)),
            pl.BlockSpec((F, F), lambda i: (0, 0)),
            pl.BlockSpec((1, F), lambda i: (0, 0)),
            pl.BlockSpec((F, F), lambda i: (0, 0)),
            pl.BlockSpec((F, 1), lambda i: (0, 0)),
        ],
        out_specs=pl.BlockSpec((NS * RB, 1), lambda i: (i, 0)),
        out_shape=jax.ShapeDtypeStruct((N, 1), jnp.float32),
        scratch_shapes=[
            pltpu.VMEM((N, F), jnp.float32),
            pltpu.VMEM((F, 1), jnp.float32),
        ],
        compiler_params=pltpu.CompilerParams(
            dimension_semantics=("parallel",)),
    )(adj, adj, x, W1, b1_2d, W2, Wc)

    c = (b2.reshape(1, F) @ Wc + bc).reshape(1, 1)

    label = pl.pallas_call(
        _p2_body,
        grid=(NSTEP,),
        in_specs=adj_specs + [
            pl.BlockSpec((N, 1), lambda i: (0, 0)),
            pl.BlockSpec((1, 1), lambda i: (0, 0), memory_space=pltpu.SMEM),
        ],
        out_specs=pl.BlockSpec((NS * RB, 1), lambda i: (i, 0)),
        out_shape=jax.ShapeDtypeStruct((N, 1), jnp.float32),
        compiler_params=pltpu.CompilerParams(
            dimension_semantics=("parallel",)),
    )(adj, adj, u, c)
    return label


# speedup vs baseline: 1.1000x; 1.0201x over previous
import jax
import jax.numpy as jnp
from jax.experimental import pallas as pl
from jax.experimental.pallas import tpu as pltpu

N = 10000
F = 128
NSTEP = 25
STREAMS = ((0, 200), (5000, 200))


def _body(*refs):
    ns = len(STREAMS)
    adj_refs = refs[:ns]
    (x_ref, W1_ref, b1_ref, W2_ref, b2_ref, Wc_ref, bc_ref,
     out_ref, u_ref, s1_ref, v_ref) = refs[ns:]
    i = pl.program_id(0)

    @pl.when(i == 0)
    def _init():
        s1_ref[...] = jnp.dot(x_ref[...], W1_ref[...],
                              preferred_element_type=jnp.float32)
        v_ref[...] = jnp.dot(W2_ref[...], Wc_ref[...],
                             preferred_element_type=jnp.float32)

    @pl.when(i < NSTEP)
    def _phase1():
        for (start, rb), a_ref in zip(STREAMS, adj_refs):
            z = jnp.dot(a_ref[...], s1_ref[...],
                        preferred_element_type=jnp.float32)
            z = jnp.maximum(z + b1_ref[...], 0.0)
            u_ref[pl.ds(start + i * rb, rb), :] = jnp.dot(
                z, v_ref[...], preferred_element_type=jnp.float32)

    @pl.when(i >= NSTEP)
    def _phase2():
        c = jnp.dot(b2_ref[...], Wc_ref[...],
                    preferred_element_type=jnp.float32) + bc_ref[...]
        j = i - NSTEP
        for (start, rb), a_ref in zip(STREAMS, adj_refs):
            out_ref[pl.ds(start + j * rb, rb), :] = jnp.dot(
                a_ref[...], u_ref[...],
                preferred_element_type=jnp.float32) + c


def kernel(adj, x, W1, b1, W2, b2, Wc, bc, We1, be1, We2, be2, Wfc, bfc):
    del We1, be1, We2, be2, Wfc, bfc
    b1_2d = b1.reshape(1, F)
    b2_2d = b2.reshape(1, F)
    bc_2d = bc.reshape(1, 1)

    adj_specs = [
        pl.BlockSpec((rb, N),
                     lambda i, st=start, r=rb: (st // r + (i % NSTEP), 0))
        for start, rb in STREAMS
    ]
    label = pl.pallas_call(
        _body,
        grid=(2 * NSTEP,),
        in_specs=adj_specs + [
            pl.BlockSpec((N, F), lambda i: (0, 0)),
            pl.BlockSpec((F, F), lambda i: (0, 0)),
            pl.BlockSpec((1, F), lambda i: (0, 0)),
            pl.BlockSpec((F, F), lambda i: (0, 0)),
            pl.BlockSpec((1, F), lambda i: (0, 0)),
            pl.BlockSpec((F, 1), lambda i: (0, 0)),
            pl.BlockSpec((1, 1), lambda i: (0, 0)),
        ],
        out_specs=pl.BlockSpec((N, 1), lambda i: (0, 0)),
        out_shape=jax.ShapeDtypeStruct((N, 1), jnp.float32),
        scratch_shapes=[
            pltpu.VMEM((N, 1), jnp.float32),
            pltpu.VMEM((N, F), jnp.float32),
            pltpu.VMEM((F, 1), jnp.float32),
        ],
    )(*([adj] * len(STREAMS)), x, W1, b1_2d, W2, b2_2d, Wc, bc_2d)
    return label


# speedup vs baseline: 1.1286x; 1.0260x over previous
import jax
import jax.numpy as jnp
from jax.experimental import pallas as pl
from jax.experimental.pallas import tpu as pltpu

N = 10000
F = 128
NS = 5
RB = 80
NSTEP = N // (NS * RB)


def _body(*refs):
    adj_refs = refs[:NS]
    (x_ref, W1_ref, b1_ref, W2_ref, b2_ref, Wc_ref, bc_ref,
     out_ref, u_ref, s1_ref, v_ref) = refs[NS:]
    i = pl.program_id(0)

    @pl.when(i == 0)
    def _init():
        s1_ref[...] = jnp.dot(x_ref[...], W1_ref[...],
                              preferred_element_type=jnp.float32)
        v_ref[...] = jnp.dot(W2_ref[...], Wc_ref[...],
                             preferred_element_type=jnp.float32)

    @pl.when(i < NSTEP)
    def _phase1():
        base = i * NS * RB
        for s, a_ref in enumerate(adj_refs):
            z = jnp.dot(a_ref[...], s1_ref[...],
                        preferred_element_type=jnp.float32)
            z = jnp.maximum(z + b1_ref[...], 0.0)
            u_ref[pl.ds(base + s * RB, RB), :] = jnp.dot(
                z, v_ref[...], preferred_element_type=jnp.float32)

    @pl.when(i >= NSTEP)
    def _phase2():
        c = jnp.dot(b2_ref[...], Wc_ref[...],
                    preferred_element_type=jnp.float32) + bc_ref[...]
        base = (i - NSTEP) * NS * RB
        for s, a_ref in enumerate(adj_refs):
            out_ref[pl.ds(base + s * RB, RB), :] = jnp.dot(
                a_ref[...], u_ref[...],
                preferred_element_type=jnp.float32) + c


def kernel(adj, x, W1, b1, W2, b2, Wc, bc, We1, be1, We2, be2, Wfc, bfc):
    del We1, be1, We2, be2, Wfc, bfc
    b1_2d = b1.reshape(1, F)
    b2_2d = b2.reshape(1, F)
    bc_2d = bc.reshape(1, 1)

    adj_specs = [
        pl.BlockSpec((RB, N), lambda i, s=s: (NS * (i % NSTEP) + s, 0))
        for s in range(NS)
    ]
    label = pl.pallas_call(
        _body,
        grid=(2 * NSTEP,),
        in_specs=adj_specs + [
            pl.BlockSpec((N, F), lambda i: (0, 0)),
            pl.BlockSpec((F, F), lambda i: (0, 0)),
            pl.BlockSpec((1, F), lambda i: (0, 0)),
            pl.BlockSpec((F, F), lambda i: (0, 0)),
            pl.BlockSpec((1, F), lambda i: (0, 0)),
            pl.BlockSpec((F, 1), lambda i: (0, 0)),
            pl.BlockSpec((1, 1), lambda i: (0, 0)),
        ],
        out_specs=pl.BlockSpec((N, 1), lambda i: (0, 0)),
        out_shape=jax.ShapeDtypeStruct((N, 1), jnp.float32),
        scratch_shapes=[
            pltpu.VMEM((N, 1), jnp.float32),
            pltpu.VMEM((N, F), jnp.float32),
            pltpu.VMEM((F, 1), jnp.float32),
        ],
    )(*([adj] * NS), x, W1, b1_2d, W2, b2_2d, Wc, bc_2d)
    return label
